# baseline (speedup 1.0000x reference)
import functools

import jax
import jax.numpy as jnp
from jax import lax
from jax.experimental import pallas as pl
from jax.experimental.pallas import tpu as pltpu
from jax.experimental.pallas import tpu_sc as plsc

_B = 64
_PERIODS = 128
_ROW_F = 384
_SEG_A = 48
_SEG_B = 80
_N_ROWS = _B * _PERIODS
_NW = 32
_ROWS_PER_W = _N_ROWS // _NW
_GROUPS = _ROWS_PER_W // 16
_BUF_W = _ROWS_PER_W * _ROW_F
_OUT_W = _ROWS_PER_W * 6
_UNROLL = 8


def _sc_body(x_hbm, o_hbm, buf, out_v):
    cid = lax.axis_index("c")
    sid = lax.axis_index("s")
    w = sid * 2 + cid
    pltpu.sync_copy(x_hbm.at[pl.ds(w * _BUF_W, _BUF_W)], buf)

    iota = lax.iota(jnp.int32, 16)
    row_base = iota * _ROW_F
    out_base = iota * 6
    zero = jnp.zeros((16,), jnp.float32)

    def seg_sums(gbase, start, n_atoms):
        def body(k, accs):
            a0, a1, a2 = accs
            p = start + k * (3 * _UNROLL)
            for u in range(_UNROLL):
                q = p + 3 * u
                a0 = a0 + plsc.load_gather(buf, [gbase + q])
                a1 = a1 + plsc.load_gather(buf, [gbase + (q + 1)])
                a2 = a2 + plsc.load_gather(buf, [gbase + (q + 2)])
            return a0, a1, a2
        return lax.fori_loop(0, n_atoms // _UNROLL, body, (zero, zero, zero))

    def group(g, carry):
        gbase = row_base + g * (16 * _ROW_F)
        a0, a1, a2 = seg_sums(gbase, 0, _SEG_A)
        b0, b1, b2 = seg_sums(gbase, 3 * _SEG_A, _SEG_B)
        obase = out_base + g * 96
        inv_a = jnp.float32(1.0 / _SEG_A)
        inv_b = jnp.float32(1.0 / _SEG_B)
        plsc.store_scatter(out_v, [obase], a0 * inv_a)
        plsc.store_scatter(out_v, [obase + 1], a1 * inv_a)
        plsc.store_scatter(out_v, [obase + 2], a2 * inv_a)
        plsc.store_scatter(out_v, [obase + 3], b0 * inv_b)
        plsc.store_scatter(out_v, [obase + 4], b1 * inv_b)
        plsc.store_scatter(out_v, [obase + 5], b2 * inv_b)
        return carry

    lax.fori_loop(0, _GROUPS, group, 0)
    pltpu.sync_copy(out_v, o_hbm.at[pl.ds(w * _OUT_W, _OUT_W)])


def kernel(inputs):
    x = inputs.reshape(-1)
    mesh = plsc.VectorSubcoreMesh(core_axis_name="c", subcore_axis_name="s")
    run = pl.kernel(
        _sc_body,
        out_type=jax.ShapeDtypeStruct((_N_ROWS * 6,), jnp.float32),
        mesh=mesh,
        scratch_types=[
            pltpu.VMEM((_BUF_W,), jnp.float32),
            pltpu.VMEM((_OUT_W,), jnp.float32),
        ],
        compiler_params=pltpu.CompilerParams(needs_layout_passes=False),
    )
    return run(x).reshape(_B, 2 * _PERIODS, 3)


# speedup vs baseline: 31.5874x; 31.5874x over previous
import jax
import jax.numpy as jnp
from jax import lax
from jax.experimental import pallas as pl
from jax.experimental.pallas import tpu as pltpu
from jax.experimental.pallas import tpu_sc as plsc

_B = 64
_PERIODS = 128
_SEG_A = 48
_SEG_B = 80
_NW = 32
_TILES = 3 * (_B // 8) * _PERIODS
_CHUNK_T = 8
_CHUNKS = _TILES // _CHUNK_T
_CHUNKS_PER_W = _CHUNKS // _NW
_UNROLL = 8


def _sc_body(x_hbm, o_hbm, buf, out_v):
    cid = lax.axis_index("c")
    sid = lax.axis_index("s")
    w = sid * 2 + cid

    iota = lax.iota(jnp.int32, 16)
    row_idx = jnp.remainder(iota, 8)
    pair_col = (iota // 8) * 128
    obase = (iota // 8) * 16 + jnp.remainder(iota, 8) * 2
    zero = jnp.zeros((16,), jnp.float32)
    inv_a = jnp.float32(1.0 / _SEG_A)
    inv_b = jnp.float32(1.0 / _SEG_B)

    def seg_sums(col0, start, n_atoms):
        def body(k, acc):
            q = start + k * _UNROLL
            for u in range(_UNROLL):
                acc = acc + plsc.load_gather(buf, [row_idx, col0 + (q + u)])
            return acc
        return lax.fori_loop(0, n_atoms // _UNROLL, body, zero)

    def chunk(j, carry):
        kg = w * _CHUNKS_PER_W + j
        c = kg // 128
        rem = kg - c * 128
        tr = rem // 16
        cb = rem - tr * 16
        pltpu.sync_copy(
            x_hbm.at[c, pl.ds(tr * 8, 8), pl.ds(cb * 1024, 1024)], buf)
        for tp in range(_CHUNK_T // 2):
            col0 = pair_col + tp * 256
            acc_a = seg_sums(col0, 0, _SEG_A)
            acc_b = seg_sums(col0, _SEG_A, _SEG_B)
            out0 = obase + (j * _CHUNK_T + tp * 2) * 16
            plsc.store_scatter(out_v, [out0], acc_a * inv_a)
            plsc.store_scatter(out_v, [out0 + 1], acc_b * inv_b)
        return carry

    lax.fori_loop(0, _CHUNKS_PER_W, chunk, 0)
    n_out = _CHUNKS_PER_W * _CHUNK_T * 16
    pltpu.sync_copy(out_v, o_hbm.at[pl.ds(w * n_out, n_out)])


def kernel(inputs):
    xt = jnp.transpose(inputs, (2, 0, 1))
    mesh = plsc.VectorSubcoreMesh(core_axis_name="c", subcore_axis_name="s")
    run = pl.kernel(
        _sc_body,
        out_type=jax.ShapeDtypeStruct((_TILES * 16,), jnp.float32),
        mesh=mesh,
        scratch_types=[
            pltpu.VMEM((8, 1024), jnp.float32),
            pltpu.VMEM((_CHUNKS_PER_W * _CHUNK_T * 16,), jnp.float32),
        ],
        compiler_params=pltpu.CompilerParams(needs_layout_passes=False),
    )
    out = run(xt)
    out = out.reshape(3, _B // 8, _PERIODS, 8, 2)
    out = out.transpose(1, 3, 2, 4, 0)
    return out.reshape(_B, 2 * _PERIODS, 3)


# speedup vs baseline: 54.0829x; 1.7122x over previous
import jax
import jax.numpy as jnp
from jax import lax
from jax.experimental import pallas as pl
from jax.experimental.pallas import tpu as pltpu
from jax.experimental.pallas import tpu_sc as plsc

_B = 64
_PERIODS = 128
_SEG_A = 48
_SEG_B = 80
_NW = 32
_TILES = 3 * (_B // 8) * _PERIODS
_CHUNK_T = 8
_CHUNKS = _TILES // _CHUNK_T
_CHUNKS_PER_W = _CHUNKS // _NW
_UNROLL = 16


def _sc_body(x_hbm, o_hbm, buf0, buf1, outc0, outc1, sem0, sem1, osem0, osem1):
    cid = lax.axis_index("c")
    sid = lax.axis_index("s")
    w = sid * 2 + cid

    iota = lax.iota(jnp.int32, 16)
    row_idx = jnp.remainder(iota, 8)
    pair_col = (iota // 8) * 128
    pair_out = (iota // 8) * 2
    zero = jnp.zeros((16,), jnp.float32)
    inv_a = jnp.float32(1.0 / _SEG_A)
    inv_b = jnp.float32(1.0 / _SEG_B)

    bufs = (buf0, buf1)
    sems = (sem0, sem1)
    outcs = (outc0, outc1)
    osems = (osem0, osem1)

    def chunk_coords(j):
        kg = w * _CHUNKS_PER_W + j
        c = kg // 128
        rem = kg - c * 128
        tr = rem // 16
        cb = rem - tr * 16
        return c, tr, cb

    def chunk_slice(j):
        c, tr, cb = chunk_coords(j)
        return x_hbm.at[c, pl.ds(tr * 8, 8), pl.ds(cb * 1024, 1024)]

    def compute(buf, outc):
        def pair(tp, carry):
            def seg(k, st):
                colv, even, odd = st
                for u in range(0, _UNROLL, 2):
                    even = even + plsc.load_gather(buf, [row_idx, colv + u])
                    odd = odd + plsc.load_gather(buf, [row_idx, colv + (u + 1)])
                return colv + _UNROLL, even, odd
            col0 = pair_col + tp * 256
            colv, ae, ao = lax.fori_loop(0, _SEG_A // _UNROLL, seg,
                                         (col0, zero, zero))
            _, be, bo = lax.fori_loop(0, _SEG_B // _UNROLL, seg,
                                      (colv, zero, zero))
            ocol = pair_out + tp * 4
            plsc.store_scatter(outc, [row_idx, ocol], (ae + ao) * inv_a)
            plsc.store_scatter(outc, [row_idx, ocol + 1], (be + bo) * inv_b)
            return carry
        lax.fori_loop(0, _CHUNK_T // 2, pair, 0)

    in_copies = [
        pltpu.async_copy(chunk_slice(0), buf0, sem0),
        pltpu.async_copy(chunk_slice(1), buf1, sem1),
    ]
    out_copies = [[], []]
    for j in range(_CHUNKS_PER_W):
        p = j % 2
        in_copies[p].wait()
        for cp in out_copies[p]:
            cp.wait()
        compute(bufs[p], outcs[p])
        if j + 2 < _CHUNKS_PER_W:
            in_copies[p] = pltpu.async_copy(chunk_slice(j + 2), bufs[p], sems[p])
        c, tr, cb = chunk_coords(j)
        base = (c * 16 + tr * 2 + cb // 8) * 1024 + (cb % 8) * 16
        out_copies[p] = [
            pltpu.async_copy(outcs[p].at[r], o_hbm.at[pl.ds(base + r * 128, 16)],
                             osems[p])
            for r in range(8)
        ]
    for p in range(2):
        for cp in out_copies[p]:
            cp.wait()


def kernel(inputs):
    xt = jnp.transpose(inputs, (2, 0, 1))
    mesh = plsc.VectorSubcoreMesh(core_axis_name="c", subcore_axis_name="s")
    run = pl.kernel(
        _sc_body,
        out_type=jax.ShapeDtypeStruct((_TILES // _B * 1024,), jnp.float32),
        mesh=mesh,
        scratch_types=[
            pltpu.VMEM((8, 1024), jnp.float32),
            pltpu.VMEM((8, 1024), jnp.float32),
            pltpu.VMEM((8, 16), jnp.float32),
            pltpu.VMEM((8, 16), jnp.float32),
            pltpu.SemaphoreType.DMA,
            pltpu.SemaphoreType.DMA,
            pltpu.SemaphoreType.DMA,
            pltpu.SemaphoreType.DMA,
        ],
        compiler_params=pltpu.CompilerParams(needs_layout_passes=False),
    )
    out = run(xt)
    out = out.reshape(3, _B // 8, 2, 8, 128)
    out = out.transpose(1, 3, 2, 4, 0)
    return out.reshape(_B, 2 * _PERIODS, 3)
